# baseline (speedup 1.0000x reference)
import functools

import jax
import jax.numpy as jnp
from jax import lax
from jax.experimental import pallas as pl
from jax.experimental.pallas import tpu as pltpu
from jax.experimental.pallas import tpu_sc as plsc

N = 512
E = 4096
B = 64
ASL = 243
HID = 64

_NC = 2
_NS = 16
_NW = _NC * _NS
_EPW = E // _NW
NP = 128

_f32 = jnp.float32


def _sc1_body(src_hbm, dst_hbm, x_hbm, ea_hbm, zeros_hbm, outx_hbm,
              oute_hbm, sidx_v, didx_v, xrows_v, erows_v, accx_sh, acce_sh, sem):
    cid = lax.axis_index("c")
    sid = lax.axis_index("s")
    base = (cid * _NS + sid) * _EPW
    pltpu.sync_copy(zeros_hbm.at[pl.ds(sid * 32, 32)], accx_sh.at[pl.ds(sid * 32, 32)])
    pltpu.sync_copy(zeros_hbm.at[pl.ds(sid * 32, 32)], acce_sh.at[pl.ds(sid * 32, 32)])
    pltpu.sync_copy(src_hbm.at[pl.ds(base, _EPW)], sidx_v)
    pltpu.sync_copy(dst_hbm.at[pl.ds(base, _EPW)], didx_v)
    pltpu.async_copy(x_hbm.at[sidx_v], xrows_v, sem).wait()
    pltpu.sync_copy(ea_hbm.at[pl.ds(base, _EPW)], erows_v)
    plsc.subcore_barrier()
    pltpu.sync_copy(xrows_v, accx_sh.at[didx_v], add=True)
    pltpu.sync_copy(erows_v, acce_sh.at[didx_v], add=True)
    plsc.subcore_barrier()
    pltpu.sync_copy(accx_sh.at[pl.ds(sid * 32, 32)], outx_hbm.at[cid, pl.ds(sid * 32, 32)])
    pltpu.sync_copy(acce_sh.at[pl.ds(sid * 32, 32)], oute_hbm.at[cid, pl.ds(sid * 32, 32)])


@functools.cache
def _get_sc1():
    return pl.kernel(
        _sc1_body,
        out_type=[jax.ShapeDtypeStruct((_NC, N, NP), _f32),
                  jax.ShapeDtypeStruct((_NC, N, NP), _f32)],
        mesh=plsc.VectorSubcoreMesh(core_axis_name="c", subcore_axis_name="s"),
        scratch_types=[
            pltpu.VMEM((_EPW,), jnp.int32),
            pltpu.VMEM((_EPW,), jnp.int32),
            pltpu.VMEM((_EPW, NP), _f32),
            pltpu.VMEM((_EPW, NP), _f32),
            pltpu.VMEM_SHARED((N, NP), _f32),
            pltpu.VMEM_SHARED((N, NP), _f32),
            pltpu.SemaphoreType.DMA,
        ],
    )


def _tc2_body(x_ref, accx_ref, acce_ref, wself_ref, wmsgp_ref, wedgep_ref, bgcn_ref,
              gid_ref, f1w_ref, f1b_ref, f2w_ref, f2b_ref,
              atop_ref, abot_ref, ab2_ref, fw_ref, fb_ref,
              slab_ref, ro_ref):
    aggx = accx_ref[0] + accx_ref[1]
    agge = acce_ref[0] + acce_ref[1]
    pre = (jnp.dot(x_ref[...], wself_ref[...], preferred_element_type=_f32)
           + jnp.dot(aggx, wmsgp_ref[...], preferred_element_type=_f32)
           + jnp.dot(agge, wedgep_ref[...], preferred_element_type=_f32)
           + bgcn_ref[...])
    h = jnp.maximum(pre, 0.0)
    gi = lax.broadcasted_iota(jnp.int32, (B, N), 0)
    pool = (gid_ref[...] == gi).astype(_f32)
    r0 = jnp.dot(pool, h, preferred_element_type=_f32)
    r1 = jnp.maximum(jnp.dot(r0, f1w_ref[...], preferred_element_type=_f32)
                     + f1b_ref[...], 0.0)
    ro_ref[...] = jnp.dot(r1, f2w_ref[...], preferred_element_type=_f32) + f2b_ref[...]
    P = jnp.dot(h[0:8], atop_ref[...], preferred_element_type=_f32)
    Q = jnp.dot(h, abot_ref[...], preferred_element_type=_f32) + ab2_ref[...]
    fw = fw_ref[...]
    fb = fb_ref[...]
    for i in range(8):
        hid = jnp.maximum(Q + P[i:i + 1], 0.0)
        slab_ref[i] = jnp.dot(hid, fw, preferred_element_type=_f32) + fb


def _tc2(x, accx, acce, wself, wmsgp, wedgep, bgcn2d, gid2d,
         f1w, f1b, f2w, f2b, atop, abot, ab2, fw, fb):
    return pl.pallas_call(
        _tc2_body,
        out_shape=[
            jax.ShapeDtypeStruct((8, N, 3), _f32),
            jax.ShapeDtypeStruct((B, 1), _f32),
        ],
    )(x, accx, acce, wself, wmsgp, wedgep, bgcn2d, gid2d,
      f1w, f1b, f2w, f2b, atop, abot, ab2, fw, fb)


def _sc2_graph(g, rowpad_hbm, impad_hbm, out_hbm, row_v, im_v, out_v):
    pltpu.sync_copy(rowpad_hbm.at[g], row_v)
    pltpu.sync_copy(impad_hbm.at[g], im_v)
    lane = lax.broadcasted_iota(jnp.int32, (16,), 0)
    neg = jnp.float32(-3e38)
    mx = jnp.float32(-3e38)
    for c in range(16):
        k0 = c * 16
        valid = (lane + k0) < ASL
        idx = jnp.where(valid, im_v[pl.ds(k0, 16)], 0)
        vals = plsc.load_gather(row_v, [idx])
        fap = jnp.where(valid, vals, neg)
        out_v[pl.ds(k0, 16)] = fap
        mx = jnp.maximum(mx, jnp.max(fap))
    tot = jnp.float32(0.0)
    for c in range(16):
        k0 = c * 16
        valid = (lane + k0) < ASL
        e = jnp.where(valid, jnp.exp(out_v[pl.ds(k0, 16)] - mx), 0.0)
        out_v[pl.ds(k0, 16)] = e
        tot = tot + jnp.sum(e)
    tot_vec = jnp.broadcast_to(tot, (16,))
    for c in range(16):
        k0 = c * 16
        out_v[pl.ds(k0, 16)] = out_v[pl.ds(k0, 16)] / tot_vec
    pltpu.sync_copy(out_v, out_hbm.at[g])


def _sc2_body(rowpad_hbm, impad_hbm, out_hbm, row_v, im_v, out_v):
    cid = lax.axis_index("c")
    sid = lax.axis_index("s")
    w = cid * _NS + sid
    for t in range(B // _NW):
        _sc2_graph(w * (B // _NW) + t, rowpad_hbm, impad_hbm,
                   out_hbm, row_v, im_v, out_v)


@functools.cache
def _get_sc2():
    return pl.kernel(
        _sc2_body,
        out_type=jax.ShapeDtypeStruct((B, 256), _f32),
        mesh=plsc.VectorSubcoreMesh(core_axis_name="c", subcore_axis_name="s"),
        compiler_params=pltpu.CompilerParams(needs_layout_passes=False),
        scratch_types=[
            pltpu.VMEM((256,), _f32),
            pltpu.VMEM((256,), jnp.int32),
            pltpu.VMEM((256,), _f32),
        ],
    )


def kernel(x, edge_attr, len_vec, mask, W_self, W_msg, W_edge, b_gcn,
           fcv1_W, fcv1_b, fcv2_W, fcv2_b, action2_W, action2_b,
           final_W, final_b, edge_index, graph_ids, num_nodes, indexmask):
    src = edge_index[0]
    dst = edge_index[1]
    xpad = jnp.pad(x, ((0, 0), (0, NP - HID)))
    eapad = jnp.pad(edge_attr, ((0, 0), (0, NP - edge_attr.shape[1])))
    zeros = jnp.zeros((N, NP), _f32)
    accx, acce = _get_sc1()(src, dst, xpad, eapad, zeros)
    slab, readout = _tc2(
        x, accx, acce, W_self,
        jnp.pad(W_msg, ((0, NP - HID), (0, 0))),
        jnp.pad(W_edge, ((0, NP - W_edge.shape[0]), (0, 0))),
        b_gcn.reshape(1, HID), graph_ids.reshape(1, N),
        fcv1_W, fcv1_b.reshape(1, -1), fcv2_W, fcv2_b.reshape(1, 1),
        action2_W[:HID], action2_W[HID:], action2_b.reshape(1, HID),
        final_W, final_b.reshape(1, 3))
    row = slab.reshape(B, 192)
    rowpad = jnp.pad(row, ((0, 0), (0, 256 - 192)))
    impad = jnp.pad(indexmask, ((0, 0), (0, 256 - ASL)))
    probs = _get_sc2()(rowpad, impad)
    return probs[:, :ASL], readout


# speedup vs baseline: 1.0644x; 1.0644x over previous
import functools

import jax
import jax.numpy as jnp
from jax import lax
from jax.experimental import pallas as pl
from jax.experimental.pallas import tpu as pltpu
from jax.experimental.pallas import tpu_sc as plsc

N = 512
E = 4096
B = 64
ASL = 243
HID = 64

_NC = 2
_NS = 16
_NW = _NC * _NS
_EPW = E // _NW
NP = 128

_f32 = jnp.float32


def _sc1_body(src_hbm, dst_hbm, x_hbm, ea_hbm, zeros_hbm, outx_hbm,
              oute_hbm, sidx_v, didx_v, xrows_v, erows_v, accx_sh, acce_sh,
              sem_i, sem_b, sem_g):
    cid = lax.axis_index("c")
    sid = lax.axis_index("s")
    base = (cid * _NS + sid) * _EPW
    rows = pl.ds(sid * 32, 32)
    ci0 = pltpu.async_copy(src_hbm.at[pl.ds(base, _EPW)], sidx_v, sem_i)
    ci1 = pltpu.async_copy(dst_hbm.at[pl.ds(base, _EPW)], didx_v, sem_i)
    cz0 = pltpu.async_copy(zeros_hbm.at[rows], accx_sh.at[rows], sem_b)
    cz1 = pltpu.async_copy(zeros_hbm.at[rows], acce_sh.at[rows], sem_b)
    ce = pltpu.async_copy(ea_hbm.at[pl.ds(base, _EPW)], erows_v, sem_b)
    ci0.wait()
    ci1.wait()
    cg = pltpu.async_copy(x_hbm.at[sidx_v], xrows_v, sem_g)
    cz0.wait()
    cz1.wait()
    ce.wait()
    cg.wait()
    plsc.subcore_barrier()
    pltpu.sync_copy(xrows_v, accx_sh.at[didx_v], add=True)
    pltpu.sync_copy(erows_v, acce_sh.at[didx_v], add=True)
    plsc.subcore_barrier()
    pltpu.sync_copy(accx_sh.at[rows], outx_hbm.at[cid, rows])
    pltpu.sync_copy(acce_sh.at[rows], oute_hbm.at[cid, rows])


@functools.cache
def _get_sc1():
    return pl.kernel(
        _sc1_body,
        out_type=[jax.ShapeDtypeStruct((_NC, N, NP), _f32),
                  jax.ShapeDtypeStruct((_NC, N, NP), _f32)],
        mesh=plsc.VectorSubcoreMesh(core_axis_name="c", subcore_axis_name="s"),
        scratch_types=[
            pltpu.VMEM((_EPW,), jnp.int32),
            pltpu.VMEM((_EPW,), jnp.int32),
            pltpu.VMEM((_EPW, NP), _f32),
            pltpu.VMEM((_EPW, NP), _f32),
            pltpu.VMEM_SHARED((N, NP), _f32),
            pltpu.VMEM_SHARED((N, NP), _f32),
            pltpu.SemaphoreType.DMA,
            pltpu.SemaphoreType.DMA,
            pltpu.SemaphoreType.DMA,
        ],
    )


def _tc2_body(x_ref, accx_ref, acce_ref, wself_ref, wmsgp_ref, wedgep_ref, bgcn_ref,
              gid_ref, f1w_ref, f1b_ref, f2w_ref, f2b_ref,
              atop_ref, abot_ref, ab2_ref, fw_ref, fb_ref,
              slab_ref, ro_ref):
    aggx = accx_ref[0] + accx_ref[1]
    agge = acce_ref[0] + acce_ref[1]
    pre = (jnp.dot(x_ref[...], wself_ref[...], preferred_element_type=_f32)
           + jnp.dot(aggx, wmsgp_ref[...], preferred_element_type=_f32)
           + jnp.dot(agge, wedgep_ref[...], preferred_element_type=_f32)
           + bgcn_ref[...])
    h = jnp.maximum(pre, 0.0)
    gi = lax.broadcasted_iota(jnp.int32, (B, N), 0)
    pool = (gid_ref[...] == gi).astype(_f32)
    r0 = jnp.dot(pool, h, preferred_element_type=_f32)
    r1 = jnp.maximum(jnp.dot(r0, f1w_ref[...], preferred_element_type=_f32)
                     + f1b_ref[...], 0.0)
    ro_ref[...] = jnp.dot(r1, f2w_ref[...], preferred_element_type=_f32) + f2b_ref[...]
    P = jnp.dot(h[0:8], atop_ref[...], preferred_element_type=_f32)
    Q = jnp.dot(h, abot_ref[...], preferred_element_type=_f32) + ab2_ref[...]
    fw = fw_ref[...]
    fb = fb_ref[...]
    for i in range(8):
        hid = jnp.maximum(Q + P[i:i + 1], 0.0)
        slab_ref[i] = jnp.dot(hid, fw, preferred_element_type=_f32) + fb


def _tc2(x, accx, acce, wself, wmsgp, wedgep, bgcn2d, gid2d,
         f1w, f1b, f2w, f2b, atop, abot, ab2, fw, fb):
    return pl.pallas_call(
        _tc2_body,
        out_shape=[
            jax.ShapeDtypeStruct((8, N, 3), _f32),
            jax.ShapeDtypeStruct((B, 1), _f32),
        ],
    )(x, accx, acce, wself, wmsgp, wedgep, bgcn2d, gid2d,
      f1w, f1b, f2w, f2b, atop, abot, ab2, fw, fb)


def _sc2_graph(g, rowpad_hbm, impad_hbm, out_hbm, row_v, im_v, out_v):
    pltpu.sync_copy(rowpad_hbm.at[g], row_v)
    pltpu.sync_copy(impad_hbm.at[g], im_v)
    lane = lax.broadcasted_iota(jnp.int32, (16,), 0)
    neg = jnp.float32(-3e38)
    mx = jnp.float32(-3e38)
    for c in range(16):
        k0 = c * 16
        valid = (lane + k0) < ASL
        idx = jnp.where(valid, im_v[pl.ds(k0, 16)], 0)
        vals = plsc.load_gather(row_v, [idx])
        fap = jnp.where(valid, vals, neg)
        out_v[pl.ds(k0, 16)] = fap
        mx = jnp.maximum(mx, jnp.max(fap))
    tot = jnp.float32(0.0)
    for c in range(16):
        k0 = c * 16
        valid = (lane + k0) < ASL
        e = jnp.where(valid, jnp.exp(out_v[pl.ds(k0, 16)] - mx), 0.0)
        out_v[pl.ds(k0, 16)] = e
        tot = tot + jnp.sum(e)
    tot_vec = jnp.broadcast_to(tot, (16,))
    for c in range(16):
        k0 = c * 16
        out_v[pl.ds(k0, 16)] = out_v[pl.ds(k0, 16)] / tot_vec
    pltpu.sync_copy(out_v, out_hbm.at[g])


def _sc2_body(rowpad_hbm, impad_hbm, out_hbm, row_v, im_v, out_v):
    cid = lax.axis_index("c")
    sid = lax.axis_index("s")
    w = cid * _NS + sid
    for t in range(B // _NW):
        _sc2_graph(w * (B // _NW) + t, rowpad_hbm, impad_hbm,
                   out_hbm, row_v, im_v, out_v)


@functools.cache
def _get_sc2():
    return pl.kernel(
        _sc2_body,
        out_type=jax.ShapeDtypeStruct((B, 256), _f32),
        mesh=plsc.VectorSubcoreMesh(core_axis_name="c", subcore_axis_name="s"),
        compiler_params=pltpu.CompilerParams(needs_layout_passes=False),
        scratch_types=[
            pltpu.VMEM((256,), _f32),
            pltpu.VMEM((256,), jnp.int32),
            pltpu.VMEM((256,), _f32),
        ],
    )


def kernel(x, edge_attr, len_vec, mask, W_self, W_msg, W_edge, b_gcn,
           fcv1_W, fcv1_b, fcv2_W, fcv2_b, action2_W, action2_b,
           final_W, final_b, edge_index, graph_ids, num_nodes, indexmask):
    src = edge_index[0]
    dst = edge_index[1]
    xpad = jnp.pad(x, ((0, 0), (0, NP - HID)))
    eapad = jnp.pad(edge_attr, ((0, 0), (0, NP - edge_attr.shape[1])))
    zeros = jnp.zeros((N, NP), _f32)
    accx, acce = _get_sc1()(src, dst, xpad, eapad, zeros)
    slab, readout = _tc2(
        x, accx, acce, W_self,
        jnp.pad(W_msg, ((0, NP - HID), (0, 0))),
        jnp.pad(W_edge, ((0, NP - W_edge.shape[0]), (0, 0))),
        b_gcn.reshape(1, HID), graph_ids.reshape(1, N),
        fcv1_W, fcv1_b.reshape(1, -1), fcv2_W, fcv2_b.reshape(1, 1),
        action2_W[:HID], action2_W[HID:], action2_b.reshape(1, HID),
        final_W, final_b.reshape(1, 3))
    row = slab.reshape(B, 192)
    rowpad = jnp.pad(row, ((0, 0), (0, 256 - 192)))
    impad = jnp.pad(indexmask, ((0, 0), (0, 256 - ASL)))
    probs = _get_sc2()(rowpad, impad)
    return probs[:, :ASL], readout


# speedup vs baseline: 1.1035x; 1.0367x over previous
import functools

import jax
import jax.numpy as jnp
from jax import lax
from jax.experimental import pallas as pl
from jax.experimental.pallas import tpu as pltpu
from jax.experimental.pallas import tpu_sc as plsc

N = 512
E = 4096
B = 64
ASL = 243
HID = 64

_NC = 2
_NS = 16
_NW = _NC * _NS
_EPW = E // _NW
NP = 128

_f32 = jnp.float32


def _sc1_body(src_hbm, dst_hbm, x_hbm, ea_hbm, zeros_hbm, outx_hbm,
              oute_hbm, sidx_v, didx_v, xrows_v, erows_v, accx_sh, acce_sh,
              sem_i, sem_b, sem_g):
    cid = lax.axis_index("c")
    sid = lax.axis_index("s")
    base = (cid * _NS + sid) * _EPW
    rows = pl.ds(sid * 32, 32)
    ci0 = pltpu.async_copy(src_hbm.at[pl.ds(base, _EPW)], sidx_v, sem_i)
    ci1 = pltpu.async_copy(dst_hbm.at[pl.ds(base, _EPW)], didx_v, sem_i)
    cz0 = pltpu.async_copy(zeros_hbm.at[rows], accx_sh.at[rows], sem_b)
    cz1 = pltpu.async_copy(zeros_hbm.at[rows], acce_sh.at[rows], sem_b)
    ce = pltpu.async_copy(ea_hbm.at[pl.ds(base, _EPW)], erows_v, sem_b)
    ci0.wait()
    ci1.wait()
    cg = pltpu.async_copy(x_hbm.at[sidx_v], xrows_v, sem_g)
    cz0.wait()
    cz1.wait()
    ce.wait()
    cg.wait()
    plsc.subcore_barrier()
    pltpu.sync_copy(xrows_v, accx_sh.at[didx_v], add=True)
    pltpu.sync_copy(erows_v, acce_sh.at[didx_v], add=True)
    plsc.subcore_barrier()
    pltpu.sync_copy(accx_sh.at[rows], outx_hbm.at[cid, rows])
    pltpu.sync_copy(acce_sh.at[rows], oute_hbm.at[cid, rows])


@functools.cache
def _get_sc1():
    return pl.kernel(
        _sc1_body,
        out_type=[jax.ShapeDtypeStruct((_NC, N, NP), _f32),
                  jax.ShapeDtypeStruct((_NC, N, NP), _f32)],
        mesh=plsc.VectorSubcoreMesh(core_axis_name="c", subcore_axis_name="s"),
        scratch_types=[
            pltpu.VMEM((_EPW,), jnp.int32),
            pltpu.VMEM((_EPW,), jnp.int32),
            pltpu.VMEM((_EPW, NP), _f32),
            pltpu.VMEM((_EPW, NP), _f32),
            pltpu.VMEM_SHARED((N, NP), _f32),
            pltpu.VMEM_SHARED((N, NP), _f32),
            pltpu.SemaphoreType.DMA,
            pltpu.SemaphoreType.DMA,
            pltpu.SemaphoreType.DMA,
        ],
    )


def _tc2_body(x_ref, accx_ref, acce_ref, wself_ref, wmsgp_ref, wedgep_ref, bgcn_ref,
              gid_ref, f1w_ref, f1b_ref, f2w_ref, f2b_ref,
              atop_ref, abot_ref, ab2_ref, fw_ref, fb_ref,
              slab_ref, ro_ref):
    aggx = accx_ref[0] + accx_ref[1]
    agge = acce_ref[0] + acce_ref[1]
    pre = (jnp.dot(x_ref[...], wself_ref[...], preferred_element_type=_f32)
           + jnp.dot(aggx, wmsgp_ref[...], preferred_element_type=_f32)
           + jnp.dot(agge, wedgep_ref[...], preferred_element_type=_f32)
           + bgcn_ref[...])
    h = jnp.maximum(pre, 0.0)
    gi = lax.broadcasted_iota(jnp.int32, (B, N), 0)
    pool = (gid_ref[...] == gi).astype(_f32)
    r0 = jnp.dot(pool, h, preferred_element_type=_f32)
    r1 = jnp.maximum(jnp.dot(r0, f1w_ref[...], preferred_element_type=_f32)
                     + f1b_ref[...], 0.0)
    ro_ref[...] = jnp.dot(r1, f2w_ref[...], preferred_element_type=_f32) + f2b_ref[...]
    P = jnp.dot(h[0:8], atop_ref[...], preferred_element_type=_f32)
    Q = jnp.dot(h, abot_ref[...], preferred_element_type=_f32) + ab2_ref[...]
    fw = fw_ref[...]
    fb = fb_ref[...]
    for i in range(8):
        hid = jnp.maximum(Q + P[i:i + 1], 0.0)
        slab_ref[i] = jnp.dot(hid, fw, preferred_element_type=_f32) + fb


def _tc2(x, accx, acce, wself, wmsgp, wedgep, bgcn2d, gid2d,
         f1w, f1b, f2w, f2b, atop, abot, ab2, fw, fb):
    return pl.pallas_call(
        _tc2_body,
        out_shape=[
            jax.ShapeDtypeStruct((8, N, 3), _f32),
            jax.ShapeDtypeStruct((B, 1), _f32),
        ],
    )(x, accx, acce, wself, wmsgp, wedgep, bgcn2d, gid2d,
      f1w, f1b, f2w, f2b, atop, abot, ab2, fw, fb)


def _sc2_graph(t, g, out_hbm, row2_v, im2_v, out2_v, sem_o):
    lane = lax.broadcasted_iota(jnp.int32, (16,), 0)
    tvec = jnp.broadcast_to(jnp.int32(t), (16,))
    neg = jnp.float32(-3e38)
    mx = jnp.float32(-3e38)
    fap = []
    for c in range(16):
        k0 = c * 16
        valid = (lane + k0) < ASL
        idx = jnp.where(valid, im2_v[t, pl.ds(k0, 16)], 0)
        vals = plsc.load_gather(row2_v, [tvec, idx])
        f = jnp.where(valid, vals, neg)
        fap.append(f)
        mx = jnp.maximum(mx, jnp.max(f))
    tot = jnp.float32(0.0)
    es = []
    for c in range(16):
        e = jnp.exp(fap[c] - mx)
        e = jnp.where(fap[c] > neg, e, 0.0)
        es.append(e)
        tot = tot + jnp.sum(e)
    tot_vec = jnp.broadcast_to(tot, (16,))
    for c in range(16):
        out2_v[t, pl.ds(c * 16, 16)] = es[c] / tot_vec
    return pltpu.async_copy(out2_v.at[t], out_hbm.at[g], sem_o)


def _sc2_body(rowpad_hbm, impad_hbm, out_hbm, row2_v, im2_v, out2_v,
              sem0, sem1, sem_o):
    cid = lax.axis_index("c")
    sid = lax.axis_index("s")
    w = cid * _NS + sid
    g0 = w * 2
    a0 = pltpu.async_copy(rowpad_hbm.at[g0], row2_v.at[0], sem0)
    b0 = pltpu.async_copy(impad_hbm.at[g0], im2_v.at[0], sem0)
    a1 = pltpu.async_copy(rowpad_hbm.at[g0 + 1], row2_v.at[1], sem1)
    b1 = pltpu.async_copy(impad_hbm.at[g0 + 1], im2_v.at[1], sem1)
    a0.wait()
    b0.wait()
    s0 = _sc2_graph(0, g0, out_hbm, row2_v, im2_v, out2_v, sem_o)
    a1.wait()
    b1.wait()
    s1 = _sc2_graph(1, g0 + 1, out_hbm, row2_v, im2_v, out2_v, sem_o)
    s0.wait()
    s1.wait()


@functools.cache
def _get_sc2():
    return pl.kernel(
        _sc2_body,
        out_type=jax.ShapeDtypeStruct((B, 256), _f32),
        mesh=plsc.VectorSubcoreMesh(core_axis_name="c", subcore_axis_name="s"),
        compiler_params=pltpu.CompilerParams(needs_layout_passes=False),
        scratch_types=[
            pltpu.VMEM((2, 256), _f32),
            pltpu.VMEM((2, 256), jnp.int32),
            pltpu.VMEM((2, 256), _f32),
            pltpu.SemaphoreType.DMA,
            pltpu.SemaphoreType.DMA,
            pltpu.SemaphoreType.DMA,
        ],
    )


def kernel(x, edge_attr, len_vec, mask, W_self, W_msg, W_edge, b_gcn,
           fcv1_W, fcv1_b, fcv2_W, fcv2_b, action2_W, action2_b,
           final_W, final_b, edge_index, graph_ids, num_nodes, indexmask):
    src = edge_index[0]
    dst = edge_index[1]
    xpad = jnp.pad(x, ((0, 0), (0, NP - HID)))
    eapad = jnp.pad(edge_attr, ((0, 0), (0, NP - edge_attr.shape[1])))
    zeros = jnp.zeros((N, NP), _f32)
    accx, acce = _get_sc1()(src, dst, xpad, eapad, zeros)
    slab, readout = _tc2(
        x, accx, acce, W_self,
        jnp.pad(W_msg, ((0, NP - HID), (0, 0))),
        jnp.pad(W_edge, ((0, NP - W_edge.shape[0]), (0, 0))),
        b_gcn.reshape(1, HID), graph_ids.reshape(1, N),
        fcv1_W, fcv1_b.reshape(1, -1), fcv2_W, fcv2_b.reshape(1, 1),
        action2_W[:HID], action2_W[HID:], action2_b.reshape(1, HID),
        final_W, final_b.reshape(1, 3))
    row = slab.reshape(B, 192)
    rowpad = jnp.pad(row, ((0, 0), (0, 256 - 192)))
    impad = jnp.pad(indexmask, ((0, 0), (0, 256 - ASL)))
    probs = _get_sc2()(rowpad, impad)
    return probs[:, :ASL], readout
